# Initial kernel scaffold; baseline (speedup 1.0000x reference)
import jax
import jax.numpy as jnp
from jax.experimental import pallas as pl


def kernel(x, edge_index, batch, W1l, b1l, W1r, b1r, W2l, b2l, W2r, b2r, Wp1, bp1, Wp2, bp2):
    raise NotImplementedError("write your pallas kernel here")


# speedup vs baseline: 7.5633x; 7.5633x over previous
import functools

import jax
import jax.numpy as jnp
from jax import lax
from jax.experimental import pallas as pl
from jax.experimental.pallas import tpu as pltpu
from jax.experimental.pallas import tpu_sc as plsc

_N = 10000
_E = 320000
_D = 128
_OUT = 64

_NC = 2
_NS = 16
_NW = _NC * _NS
_EPW = _E // _NW
_K = 80
_NCH = _EPW // _K
_NPAD = 10240
_NPT = _NPAD // _NS


def _make_segsum(with_count):
    mesh = plsc.VectorSubcoreMesh(core_axis_name="c", subcore_axis_name="s")
    if with_count:
        out_type = (
            jax.ShapeDtypeStruct((_NC, _NPAD, _D), jnp.float32),
            jax.ShapeDtypeStruct((_NC, _NPAD), jnp.float32),
        )
    else:
        out_type = jax.ShapeDtypeStruct((_NC, _NPAD, _D), jnp.float32)

    scratch = [
        pltpu.VMEM((_NCH, _K), jnp.int32),
        pltpu.VMEM((_NCH, _K), jnp.int32),
        pltpu.VMEM((_K, _D), jnp.float32),
        pltpu.VMEM((_K,), jnp.float32),
        pltpu.VMEM_SHARED((_NPAD, _D), jnp.float32),
        pltpu.VMEM_SHARED((_NPAD,), jnp.float32),
        pltpu.SemaphoreType.DMA,
    ]

    @functools.partial(pl.kernel, mesh=mesh, out_type=out_type,
                       scratch_types=scratch)
    def seg(x_hbm, srcg, dstg, zrows, zcnt, *rest):
        if with_count:
            s_out, c_out, idxs, idxd, rows, ones_v, acc, cnt, sem = rest
        else:
            s_out, idxs, idxd, rows, ones_v, acc, cnt, sem = rest
        c = lax.axis_index("c")
        s = lax.axis_index("s")
        wid = c * _NS + s

        pltpu.sync_copy(zrows.at[pl.ds(s * _NPT, _NPT)],
                        acc.at[pl.ds(s * _NPT, _NPT)])
        if with_count:
            @pl.when(s == 0)
            def _zero_cnt():
                pltpu.sync_copy(zcnt, cnt)

            def _fill(i, carry):
                ones_v[pl.ds(i * 16, 16)] = jnp.ones((16,), jnp.float32)
                return carry
            lax.fori_loop(0, _K // 16, _fill, 0)

        pltpu.sync_copy(srcg.at[wid], idxs)
        pltpu.sync_copy(dstg.at[wid], idxd)
        plsc.subcore_barrier()

        def _body(j, carry):
            pltpu.async_copy(x_hbm.at[idxs.at[j]], rows, sem).wait()
            pltpu.sync_copy(rows, acc.at[idxd.at[j]], add=True)
            if with_count:
                pltpu.sync_copy(ones_v, cnt.at[idxd.at[j]], add=True)
            return carry
        lax.fori_loop(0, _NCH, _body, 0)
        plsc.subcore_barrier()

        pltpu.sync_copy(acc.at[pl.ds(s * _NPT, _NPT)],
                        s_out.at[c, pl.ds(s * _NPT, _NPT)])
        if with_count:
            @pl.when(s == 0)
            def _cnt_out():
                pltpu.sync_copy(cnt, c_out.at[c])

    return seg


_segsum_cnt = _make_segsum(True)
_segsum = _make_segsum(False)

_BR = 1000


def _combine(sp_ref, cn_ref):
    ssum = sp_ref[0] + sp_ref[1]
    cn = cn_ref[0] + cn_ref[1]
    return ssum / jnp.maximum(cn, 1.0)


def _sage(x, agg, wl_ref, bl_ref, wr_ref, br_ref):
    dn = (((1,), (1,)), ((), ()))
    out = (lax.dot_general(x, wl_ref[...], dn,
                           preferred_element_type=jnp.float32)
           + lax.dot_general(agg, wr_ref[...], dn,
                             preferred_element_type=jnp.float32)
           + bl_ref[...] + br_ref[...])
    nrm = jnp.sqrt(jnp.sum(out * out, axis=1, keepdims=True))
    out = out / jnp.maximum(nrm, 1e-12)
    return jnp.maximum(out, 0.0)


def _layer1_body(x_ref, sp_ref, cn_ref, wl_ref, bl_ref, wr_ref, br_ref,
                 o_ref):
    agg = _combine(sp_ref, cn_ref)
    o_ref[...] = _sage(x_ref[...], agg, wl_ref, bl_ref, wr_ref, br_ref)


def _layer2_body(h_ref, sp_ref, cn_ref, wl_ref, bl_ref, wr_ref, br_ref,
                 wp1_ref, bp1_ref, wp2_ref, bp2_ref, o_ref):
    agg = _combine(sp_ref, cn_ref)
    h2 = _sage(h_ref[...], agg, wl_ref, bl_ref, wr_ref, br_ref)
    dn = (((1,), (1,)), ((), ()))
    t = lax.dot_general(h2, wp1_ref[...], dn,
                        preferred_element_type=jnp.float32) + bp1_ref[...]
    y = lax.dot_general(t, wp2_ref[...], dn,
                        preferred_element_type=jnp.float32) + bp2_ref[...]
    m = jnp.max(y, axis=1, keepdims=True)
    z = y - m
    o_ref[...] = z - jnp.log(jnp.sum(jnp.exp(z), axis=1, keepdims=True))


def _wspec(r, c):
    return pl.BlockSpec((r, c), lambda i: (0, 0))


_ROW_SPECS = [
    pl.BlockSpec((_BR, _D), lambda i: (i, 0)),
    pl.BlockSpec((_NC, _BR, _D), lambda i: (0, i, 0)),
    pl.BlockSpec((_NC, _BR, 1), lambda i: (0, i, 0)),
]

_layer1 = pl.pallas_call(
    _layer1_body,
    grid=(_N // _BR,),
    in_specs=_ROW_SPECS + [_wspec(_D, _D), _wspec(1, _D),
                           _wspec(_D, _D), _wspec(1, _D)],
    out_specs=pl.BlockSpec((_BR, _D), lambda i: (i, 0)),
    out_shape=jax.ShapeDtypeStruct((_N, _D), jnp.float32),
)

_layer2 = pl.pallas_call(
    _layer2_body,
    grid=(_N // _BR,),
    in_specs=_ROW_SPECS + [_wspec(_D, _D), _wspec(1, _D),
                           _wspec(_D, _D), _wspec(1, _D),
                           _wspec(_D, _D), _wspec(1, _D),
                           _wspec(_OUT, _D), _wspec(1, _OUT)],
    out_specs=pl.BlockSpec((_BR, _OUT), lambda i: (i, 0)),
    out_shape=jax.ShapeDtypeStruct((_N, _OUT), jnp.float32),
)


def kernel(x, edge_index, batch, W1l, b1l, W1r, b1r, W2l, b2l, W2r, b2r,
           Wp1, bp1, Wp2, bp2):
    src = edge_index[0].reshape(_NW, _NCH, _K)
    dst = edge_index[1].reshape(_NW, _NCH, _K)
    zrows = jnp.zeros((_NPAD, _D), jnp.float32)
    zcnt = jnp.zeros((_NPAD,), jnp.float32)

    s1, cnt = _segsum_cnt(x, src, dst, zrows, zcnt)
    cnt3 = cnt.reshape(_NC, _NPAD, 1)
    h1 = _layer1(x, s1, cnt3, W1l, b1l.reshape(1, _D), W1r,
                 b1r.reshape(1, _D))
    s2 = _segsum(h1, src, dst, zrows, zcnt)
    out = _layer2(h1, s2, cnt3, W2l, b2l.reshape(1, _D), W2r,
                  b2r.reshape(1, _D), Wp1, bp1.reshape(1, _D),
                  Wp2, bp2.reshape(1, _OUT))
    return out
